# baseline (speedup 1.0000x reference)
import functools

import jax
import jax.numpy as jnp
from jax import lax
from jax.experimental import pallas as pl
from jax.experimental.pallas import tpu as pltpu
from jax.experimental.pallas import tpu_sc as plsc

N_CORES = 2
N_SUB = 16
NW = N_CORES * N_SUB

CHUNK = 80
ZCH = 80


def _sc_mesh():
    return plsc.VectorSubcoreMesh(core_axis_name="c", subcore_axis_name="s")


def _make_scatter(E, NP, D):
    e_per_w = E // NW
    n_groups = e_per_w // CHUNK
    n_pairs = n_groups // 2
    rows_per_sub = NP // N_SUB
    n_zch = rows_per_sub // ZCH

    @functools.partial(
        pl.kernel,
        out_type=jax.ShapeDtypeStruct((N_CORES, NP, D), jnp.float32),
        mesh=_sc_mesh(),
        scratch_types=[
            pltpu.VMEM_SHARED((NP, D), jnp.float32),
            pltpu.VMEM((n_groups, CHUNK), jnp.int32),
            pltpu.VMEM((CHUNK, D), jnp.float32),
            pltpu.VMEM((CHUNK, D), jnp.float32),
            pltpu.SemaphoreType.DMA,
            pltpu.SemaphoreType.DMA,
            pltpu.SemaphoreType.DMA,
            pltpu.SemaphoreType.DMA,
        ],
    )
    def scatter(msg_hbm, dst_hbm, zeros_hbm, inc_hbm, acc_sh, idx_v,
                buf0, buf1, lsem0, lsem1, ssem0, ssem1):
        cid = lax.axis_index("c")
        sid = lax.axis_index("s")
        wid = sid * N_CORES + cid
        slab = sid * rows_per_sub
        pltpu.sync_copy(zeros_hbm, buf0)
        for i in range(n_zch):
            pltpu.make_async_copy(
                buf0, acc_sh.at[pl.ds(slab + i * ZCH, ZCH)], lsem0).start()
        for i in range(n_zch):
            pltpu.make_async_copy(
                buf0, acc_sh.at[pl.ds(slab + i * ZCH, ZCH)], lsem0).wait()
        plsc.subcore_barrier()
        pltpu.sync_copy(dst_hbm.at[wid], idx_v)
        base = wid * e_per_w

        def start_load(g, buf, sem):
            pltpu.make_async_copy(
                msg_hbm.at[pl.ds(base + g * CHUNK, CHUNK)], buf, sem).start()

        def wait_load(buf, sem):
            pltpu.make_async_copy(
                msg_hbm.at[pl.ds(base, CHUNK)], buf, sem).wait()

        def start_scat(g, buf, sem):
            pltpu.make_async_copy(
                buf, acc_sh.at[idx_v.at[g]], sem).start(add=True)

        def wait_scat(g, buf, sem):
            pltpu.make_async_copy(
                buf, acc_sh.at[idx_v.at[g]], sem).wait()

        start_load(0, buf0, lsem0)

        def pair(p, carry):
            g0 = 2 * p

            @pl.when(p > 0)
            def _():
                wait_scat(g0 - 1, buf1, ssem1)

            start_load(g0 + 1, buf1, lsem1)
            wait_load(buf0, lsem0)
            start_scat(g0, buf0, ssem0)
            wait_scat(g0, buf0, ssem0)
            start_load(g0 + 2, buf0, lsem0)
            wait_load(buf1, lsem1)
            start_scat(g0 + 1, buf1, ssem1)
            return carry

        lax.fori_loop(0, n_pairs, pair, 0)
        wait_scat(n_groups - 2, buf1, ssem1)
        wait_load(buf0, lsem0)
        start_scat(n_groups - 1, buf0, ssem0)
        wait_scat(n_groups - 1, buf0, ssem0)
        plsc.subcore_barrier()
        for i in range(n_zch):
            bb = buf0 if i % 2 == 0 else buf1
            sem = lsem0 if i % 2 == 0 else lsem1
            if i >= 2:
                pltpu.make_async_copy(
                    bb, inc_hbm.at[cid, pl.ds(slab + (i - 2) * ZCH, ZCH)],
                    sem).wait()
            pltpu.sync_copy(acc_sh.at[pl.ds(slab + i * ZCH, ZCH)], bb)
            pltpu.make_async_copy(
                bb, inc_hbm.at[cid, pl.ds(slab + i * ZCH, ZCH)], sem).start()
        for i in range(n_zch - 2, n_zch):
            bb = buf0 if i % 2 == 0 else buf1
            sem = lsem0 if i % 2 == 0 else lsem1
            pltpu.make_async_copy(
                bb, inc_hbm.at[cid, pl.ds(slab + i * ZCH, ZCH)], sem).wait()

    return scatter


def _make_gather(E, NP, D):
    e_per_w = E // NW
    n_groups = e_per_w // CHUNK
    n_pairs = n_groups // 2
    rows_per_sub = NP // N_SUB
    n_zch = rows_per_sub // ZCH

    @functools.partial(
        pl.kernel,
        out_type=jax.ShapeDtypeStruct((E, D), jnp.float32),
        mesh=_sc_mesh(),
        scratch_types=[
            pltpu.VMEM_SHARED((NP, D), jnp.float32),
            pltpu.VMEM((n_groups, CHUNK), jnp.int32),
            pltpu.VMEM((CHUNK, D), jnp.float32),
            pltpu.VMEM((CHUNK, D), jnp.float32),
            pltpu.SemaphoreType.DMA,
            pltpu.SemaphoreType.DMA,
            pltpu.SemaphoreType.DMA,
            pltpu.SemaphoreType.DMA,
        ],
    )
    def gather(nt_hbm, src_hbm, out_hbm, tab_sh, idx_v, buf0, buf1,
               gsem0, gsem1, wsem0, wsem1):
        cid = lax.axis_index("c")
        sid = lax.axis_index("s")
        wid = sid * N_CORES + cid
        slab = sid * rows_per_sub
        for i in range(2):
            bb = buf0 if i % 2 == 0 else buf1
            sem = gsem0 if i % 2 == 0 else gsem1
            pltpu.make_async_copy(
                nt_hbm.at[pl.ds(slab + i * ZCH, ZCH)], bb, sem).start()
        for i in range(n_zch):
            bb = buf0 if i % 2 == 0 else buf1
            sem = gsem0 if i % 2 == 0 else gsem1
            r0 = slab + i * ZCH
            pltpu.make_async_copy(nt_hbm.at[pl.ds(r0, ZCH)], bb, sem).wait()
            pltpu.sync_copy(bb, tab_sh.at[pl.ds(r0, ZCH)])
            if i + 2 < n_zch:
                pltpu.make_async_copy(
                    nt_hbm.at[pl.ds(slab + (i + 2) * ZCH, ZCH)], bb,
                    sem).start()
        plsc.subcore_barrier()
        pltpu.sync_copy(src_hbm.at[wid], idx_v)
        base = wid * e_per_w

        def start_gat(g, buf, sem):
            pltpu.make_async_copy(
                tab_sh.at[idx_v.at[g]], buf, sem).start()

        def wait_gat(g, buf, sem):
            pltpu.make_async_copy(
                tab_sh.at[idx_v.at[g]], buf, sem).wait()

        def start_store(g, buf, sem):
            pltpu.make_async_copy(
                buf, out_hbm.at[pl.ds(base + g * CHUNK, CHUNK)], sem).start()

        def wait_store(g, buf, sem):
            pltpu.make_async_copy(
                buf, out_hbm.at[pl.ds(base + g * CHUNK, CHUNK)], sem).wait()

        start_gat(0, buf0, gsem0)

        def pair(p, carry):
            g0 = 2 * p

            @pl.when(p > 0)
            def _():
                wait_store(g0 - 1, buf1, wsem1)

            start_gat(g0 + 1, buf1, gsem1)
            wait_gat(g0, buf0, gsem0)
            start_store(g0, buf0, wsem0)
            wait_store(g0, buf0, wsem0)
            start_gat(g0 + 2, buf0, gsem0)
            wait_gat(g0 + 1, buf1, gsem1)
            start_store(g0 + 1, buf1, wsem1)
            return carry

        lax.fori_loop(0, n_pairs, pair, 0)
        wait_store(n_groups - 2, buf1, wsem1)
        wait_gat(n_groups - 1, buf0, gsem0)
        start_store(n_groups - 1, buf0, wsem0)
        wait_store(n_groups - 1, buf0, wsem0)

    return gather


def _node_term_body(atom_ref, inc_ref, wa_ref, wc_ref, b_ref, o_ref):
    inc = inc_ref[0] + inc_ref[1]
    acc = jnp.dot(atom_ref[...], wa_ref[...],
                  preferred_element_type=jnp.float32)
    acc += jnp.dot(inc, wc_ref[...], preferred_element_type=jnp.float32)
    o_ref[...] = acc + b_ref[...]


def _edge_body(msg_ref, g_ref, bond_ref, wb_ref, gam_ref, bet_ref, o_ref):
    t = g_ref[...] + jnp.dot(bond_ref[...], wb_ref[...],
                             preferred_element_type=jnp.float32)
    x = msg_ref[...] + jnp.maximum(t, 0.0)
    mu = jnp.mean(x, axis=1, keepdims=True)
    xc = x - mu
    var = jnp.mean(xc * xc, axis=1, keepdims=True)
    inv = lax.rsqrt(var + 1e-5)
    o_ref[...] = xc * inv * gam_ref[...] + bet_ref[...]


def _edge_body_aliased(prev_ref, msg_ref, g_ref, bond_ref, wb_ref, gam_ref,
                       bet_ref, o_ref):
    del prev_ref
    _edge_body(msg_ref, g_ref, bond_ref, wb_ref, gam_ref, bet_ref, o_ref)


def kernel(msg, atom, bond, src, dst, W, b, gamma, beta):
    E, D = msg.shape
    N = atom.shape[0]
    BD = bond.shape[1]
    NP = -(-N // (N_SUB * ZCH)) * (N_SUB * ZCH)

    wa_t = W[:, :D].T
    wb_t = W[:, D:D + BD].T
    wc_t = W[:, D + BD:].T
    dst_r = dst.astype(jnp.int32).reshape(NW, (E // NW) // CHUNK, CHUNK)
    src_r = src.astype(jnp.int32).reshape(NW, (E // NW) // CHUNK, CHUNK)
    zeros = jnp.zeros((ZCH, D), jnp.float32)
    atom_p = jnp.pad(atom, ((0, NP - N), (0, 0)))

    inc_part = _make_scatter(E, NP, D)(msg, dst_r, zeros)

    nblk = 1024
    node_term = pl.pallas_call(
        _node_term_body,
        grid=(NP // nblk,),
        in_specs=[
            pl.BlockSpec((nblk, D), lambda i: (i, 0)),
            pl.BlockSpec((N_CORES, nblk, D), lambda i: (0, i, 0)),
            pl.BlockSpec((D, D), lambda i: (0, 0)),
            pl.BlockSpec((D, D), lambda i: (0, 0)),
            pl.BlockSpec((1, D), lambda i: (0, 0)),
        ],
        out_specs=pl.BlockSpec((nblk, D), lambda i: (i, 0)),
        out_shape=jax.ShapeDtypeStruct((NP, D), jnp.float32),
    )(atom_p, inc_part, wa_t, wc_t, b.reshape(1, D))

    K = 5
    EC = E // K
    eblk = 2000
    spc = EC // eblk
    gather_fn = _make_gather(EC, NP, D)
    src_c = src_r.reshape(K, NW, (EC // NW) // CHUNK, CHUNK)
    gam = gamma.reshape(1, D)
    bet = beta.reshape(1, D)

    gathered = [gather_fn(node_term, src_c[k]) for k in range(K)]

    out = None
    for k in range(K):
        off = k * spc
        chunk_specs = [
            pl.BlockSpec((eblk, D), lambda i, o=off: (o + i, 0)),
            pl.BlockSpec((eblk, D), lambda i: (i, 0)),
            pl.BlockSpec((eblk, BD), lambda i, o=off: (o + i, 0)),
            pl.BlockSpec((BD, D), lambda i: (0, 0)),
            pl.BlockSpec((1, D), lambda i: (0, 0)),
            pl.BlockSpec((1, D), lambda i: (0, 0)),
        ]
        out_spec = pl.BlockSpec((eblk, D), lambda i, o=off: (o + i, 0))
        if k == 0:
            out = pl.pallas_call(
                _edge_body,
                grid=(spc,),
                in_specs=chunk_specs,
                out_specs=out_spec,
                out_shape=jax.ShapeDtypeStruct((E, D), jnp.float32),
            )(msg, gathered[k], bond, wb_t, gam, bet)
        else:
            out = pl.pallas_call(
                _edge_body_aliased,
                grid=(spc,),
                in_specs=[pl.BlockSpec(memory_space=pl.ANY)] + chunk_specs,
                out_specs=out_spec,
                out_shape=jax.ShapeDtypeStruct((E, D), jnp.float32),
                input_output_aliases={0: 0},
            )(out, msg, gathered[k], bond, wb_t, gam, bet)

    return out


# speedup vs baseline: 1.2483x; 1.2483x over previous
import functools

import jax
import jax.numpy as jnp
from jax import lax
from jax.experimental import pallas as pl
from jax.experimental.pallas import tpu as pltpu
from jax.experimental.pallas import tpu_sc as plsc

N_CORES = 2
N_SUB = 16
NW = N_CORES * N_SUB

CHUNK = 80
ZCH = 80


def _sc_mesh():
    return plsc.VectorSubcoreMesh(core_axis_name="c", subcore_axis_name="s")


def _make_scatter(E, NP, D):
    e_per_w = E // NW
    n_groups = e_per_w // CHUNK
    n_pairs = n_groups // 2
    rows_per_sub = NP // N_SUB
    n_zch = rows_per_sub // ZCH

    @functools.partial(
        pl.kernel,
        out_type=jax.ShapeDtypeStruct((N_CORES, NP, D), jnp.float32),
        mesh=_sc_mesh(),
        scratch_types=[
            pltpu.VMEM_SHARED((NP, D), jnp.float32),
            pltpu.VMEM((n_groups, CHUNK), jnp.int32),
            pltpu.VMEM((CHUNK, D), jnp.float32),
            pltpu.VMEM((CHUNK, D), jnp.float32),
            pltpu.SemaphoreType.DMA,
            pltpu.SemaphoreType.DMA,
            pltpu.SemaphoreType.DMA,
            pltpu.SemaphoreType.DMA,
        ],
    )
    def scatter(msg_hbm, dst_hbm, zeros_hbm, inc_hbm, acc_sh, idx_v,
                buf0, buf1, lsem0, lsem1, ssem0, ssem1):
        cid = lax.axis_index("c")
        sid = lax.axis_index("s")
        wid = sid * N_CORES + cid
        slab = sid * rows_per_sub
        pltpu.sync_copy(zeros_hbm, buf0)
        for i in range(n_zch):
            pltpu.make_async_copy(
                buf0, acc_sh.at[pl.ds(slab + i * ZCH, ZCH)], lsem0).start()
        for i in range(n_zch):
            pltpu.make_async_copy(
                buf0, acc_sh.at[pl.ds(slab + i * ZCH, ZCH)], lsem0).wait()
        plsc.subcore_barrier()
        pltpu.sync_copy(dst_hbm.at[wid], idx_v)
        base = wid * e_per_w

        def start_load(g, buf, sem):
            pltpu.make_async_copy(
                msg_hbm.at[pl.ds(base + g * CHUNK, CHUNK)], buf, sem).start()

        def wait_load(buf, sem):
            pltpu.make_async_copy(
                msg_hbm.at[pl.ds(base, CHUNK)], buf, sem).wait()

        def start_scat(g, buf, sem):
            pltpu.make_async_copy(
                buf, acc_sh.at[idx_v.at[g]], sem).start(add=True)

        def wait_scat(g, buf, sem):
            pltpu.make_async_copy(
                buf, acc_sh.at[idx_v.at[g]], sem).wait()

        start_load(0, buf0, lsem0)

        def pair(p, carry):
            g0 = 2 * p

            @pl.when(p > 0)
            def _():
                wait_scat(g0 - 1, buf1, ssem1)

            start_load(g0 + 1, buf1, lsem1)
            wait_load(buf0, lsem0)
            start_scat(g0, buf0, ssem0)
            wait_scat(g0, buf0, ssem0)
            start_load(g0 + 2, buf0, lsem0)
            wait_load(buf1, lsem1)
            start_scat(g0 + 1, buf1, ssem1)
            return carry

        lax.fori_loop(0, n_pairs, pair, 0)
        wait_scat(n_groups - 2, buf1, ssem1)
        wait_load(buf0, lsem0)
        start_scat(n_groups - 1, buf0, ssem0)
        wait_scat(n_groups - 1, buf0, ssem0)
        plsc.subcore_barrier()
        for i in range(n_zch):
            bb = buf0 if i % 2 == 0 else buf1
            sem = lsem0 if i % 2 == 0 else lsem1
            if i >= 2:
                pltpu.make_async_copy(
                    bb, inc_hbm.at[cid, pl.ds(slab + (i - 2) * ZCH, ZCH)],
                    sem).wait()
            pltpu.sync_copy(acc_sh.at[pl.ds(slab + i * ZCH, ZCH)], bb)
            pltpu.make_async_copy(
                bb, inc_hbm.at[cid, pl.ds(slab + i * ZCH, ZCH)], sem).start()
        for i in range(n_zch - 2, n_zch):
            bb = buf0 if i % 2 == 0 else buf1
            sem = lsem0 if i % 2 == 0 else lsem1
            pltpu.make_async_copy(
                bb, inc_hbm.at[cid, pl.ds(slab + i * ZCH, ZCH)], sem).wait()

    return scatter


def _make_gather(E, NP, D):
    e_per_w = E // NW
    n_groups = e_per_w // CHUNK
    n_pairs = n_groups // 2
    rows_per_sub = NP // N_SUB
    n_zch = rows_per_sub // ZCH

    @functools.partial(
        pl.kernel,
        out_type=jax.ShapeDtypeStruct((E, D), jnp.float32),
        mesh=_sc_mesh(),
        scratch_types=[
            pltpu.VMEM_SHARED((NP, D), jnp.float32),
            pltpu.VMEM((n_groups, CHUNK), jnp.int32),
            pltpu.VMEM((CHUNK, D), jnp.float32),
            pltpu.VMEM((CHUNK, D), jnp.float32),
            pltpu.SemaphoreType.DMA,
            pltpu.SemaphoreType.DMA,
            pltpu.SemaphoreType.DMA,
            pltpu.SemaphoreType.DMA,
        ],
    )
    def gather(nt_hbm, src_hbm, out_hbm, tab_sh, idx_v, buf0, buf1,
               gsem0, gsem1, wsem0, wsem1):
        cid = lax.axis_index("c")
        sid = lax.axis_index("s")
        wid = sid * N_CORES + cid
        slab = sid * rows_per_sub
        for i in range(2):
            bb = buf0 if i % 2 == 0 else buf1
            sem = gsem0 if i % 2 == 0 else gsem1
            pltpu.make_async_copy(
                nt_hbm.at[pl.ds(slab + i * ZCH, ZCH)], bb, sem).start()
        for i in range(n_zch):
            bb = buf0 if i % 2 == 0 else buf1
            sem = gsem0 if i % 2 == 0 else gsem1
            r0 = slab + i * ZCH
            pltpu.make_async_copy(nt_hbm.at[pl.ds(r0, ZCH)], bb, sem).wait()
            pltpu.sync_copy(bb, tab_sh.at[pl.ds(r0, ZCH)])
            if i + 2 < n_zch:
                pltpu.make_async_copy(
                    nt_hbm.at[pl.ds(slab + (i + 2) * ZCH, ZCH)], bb,
                    sem).start()
        plsc.subcore_barrier()
        pltpu.sync_copy(src_hbm.at[wid], idx_v)
        base = wid * e_per_w

        def start_gat(g, buf, sem):
            pltpu.make_async_copy(
                tab_sh.at[idx_v.at[g]], buf, sem).start()

        def wait_gat(g, buf, sem):
            pltpu.make_async_copy(
                tab_sh.at[idx_v.at[g]], buf, sem).wait()

        def start_store(g, buf, sem):
            pltpu.make_async_copy(
                buf, out_hbm.at[pl.ds(base + g * CHUNK, CHUNK)], sem).start()

        def wait_store(g, buf, sem):
            pltpu.make_async_copy(
                buf, out_hbm.at[pl.ds(base + g * CHUNK, CHUNK)], sem).wait()

        start_gat(0, buf0, gsem0)

        def pair(p, carry):
            g0 = 2 * p

            @pl.when(p > 0)
            def _():
                wait_store(g0 - 1, buf1, wsem1)

            start_gat(g0 + 1, buf1, gsem1)
            wait_gat(g0, buf0, gsem0)
            start_store(g0, buf0, wsem0)
            wait_store(g0, buf0, wsem0)
            start_gat(g0 + 2, buf0, gsem0)
            wait_gat(g0 + 1, buf1, gsem1)
            start_store(g0 + 1, buf1, wsem1)
            return carry

        lax.fori_loop(0, n_pairs, pair, 0)
        wait_store(n_groups - 2, buf1, wsem1)
        wait_gat(n_groups - 1, buf0, gsem0)
        start_store(n_groups - 1, buf0, wsem0)
        wait_store(n_groups - 1, buf0, wsem0)

    return gather


def _node_term_body(atom_ref, inc_ref, wa_ref, wc_ref, b_ref, o_ref):
    inc = inc_ref[0] + inc_ref[1]
    acc = jnp.dot(atom_ref[...], wa_ref[...],
                  preferred_element_type=jnp.float32)
    acc += jnp.dot(inc, wc_ref[...], preferred_element_type=jnp.float32)
    o_ref[...] = acc + b_ref[...]


def _edge_body(msg_ref, g_ref, bond_ref, wb_ref, gam_ref, bet_ref, o_ref):
    bw = lax.dot_general(bond_ref[...], wb_ref[...], (((0,), (0,)), ((), ())),
                         preferred_element_type=jnp.float32)
    t = g_ref[...] + bw
    x = msg_ref[...] + jnp.maximum(t, 0.0)
    mu = jnp.mean(x, axis=1, keepdims=True)
    xc = x - mu
    var = jnp.mean(xc * xc, axis=1, keepdims=True)
    inv = lax.rsqrt(var + 1e-5)
    o_ref[...] = xc * inv * gam_ref[...] + bet_ref[...]


def _edge_body_aliased(prev_ref, msg_ref, g_ref, bond_ref, wb_ref, gam_ref,
                       bet_ref, o_ref):
    del prev_ref
    _edge_body(msg_ref, g_ref, bond_ref, wb_ref, gam_ref, bet_ref, o_ref)


def kernel(msg, atom, bond, src, dst, W, b, gamma, beta):
    E, D = msg.shape
    N = atom.shape[0]
    BD = bond.shape[1]
    NP = -(-N // (N_SUB * ZCH)) * (N_SUB * ZCH)

    wa_t = W[:, :D].T
    wb_t = W[:, D:D + BD].T
    wc_t = W[:, D + BD:].T
    dst_r = dst.astype(jnp.int32).reshape(NW, (E // NW) // CHUNK, CHUNK)
    src_r = src.astype(jnp.int32).reshape(NW, (E // NW) // CHUNK, CHUNK)
    zeros = jnp.zeros((ZCH, D), jnp.float32)
    atom_p = jnp.pad(atom, ((0, NP - N), (0, 0)))

    inc_part = _make_scatter(E, NP, D)(msg, dst_r, zeros)

    nblk = 1024
    node_term = pl.pallas_call(
        _node_term_body,
        grid=(NP // nblk,),
        in_specs=[
            pl.BlockSpec((nblk, D), lambda i: (i, 0)),
            pl.BlockSpec((N_CORES, nblk, D), lambda i: (0, i, 0)),
            pl.BlockSpec((D, D), lambda i: (0, 0)),
            pl.BlockSpec((D, D), lambda i: (0, 0)),
            pl.BlockSpec((1, D), lambda i: (0, 0)),
        ],
        out_specs=pl.BlockSpec((nblk, D), lambda i: (i, 0)),
        out_shape=jax.ShapeDtypeStruct((NP, D), jnp.float32),
    )(atom_p, inc_part, wa_t, wc_t, b.reshape(1, D))

    K = 5
    EC = E // K
    eblk = 2560
    spc = EC // eblk
    gather_fn = _make_gather(EC, NP, D)
    src_c = src_r.reshape(K, NW, (EC // NW) // CHUNK, CHUNK)
    gam = gamma.reshape(1, D)
    bet = beta.reshape(1, D)

    bond_t = bond.T
    gathered = [gather_fn(node_term, src_c[k]) for k in range(K)]

    out = None
    for k in range(K):
        off = k * spc
        chunk_specs = [
            pl.BlockSpec((eblk, D), lambda i, o=off: (o + i, 0)),
            pl.BlockSpec((eblk, D), lambda i: (i, 0)),
            pl.BlockSpec((BD, eblk), lambda i, o=off: (0, o + i)),
            pl.BlockSpec((BD, D), lambda i: (0, 0)),
            pl.BlockSpec((1, D), lambda i: (0, 0)),
            pl.BlockSpec((1, D), lambda i: (0, 0)),
        ]
        out_spec = pl.BlockSpec((eblk, D), lambda i, o=off: (o + i, 0))
        if k == 0:
            out = pl.pallas_call(
                _edge_body,
                grid=(spc,),
                in_specs=chunk_specs,
                out_specs=out_spec,
                out_shape=jax.ShapeDtypeStruct((E, D), jnp.float32),
            )(msg, gathered[k], bond_t, wb_t, gam, bet)
        else:
            out = pl.pallas_call(
                _edge_body_aliased,
                grid=(spc,),
                in_specs=[pl.BlockSpec(memory_space=pl.ANY)] + chunk_specs,
                out_specs=out_spec,
                out_shape=jax.ShapeDtypeStruct((E, D), jnp.float32),
                input_output_aliases={0: 0},
            )(out, msg, gathered[k], bond_t, wb_t, gam, bet)

    return out
